# baseline (speedup 1.0000x reference)
import functools
import math

import jax
import jax.numpy as jnp
from jax import lax
from jax.experimental import pallas as pl
from jax.experimental.pallas import tpu as pltpu
from jax.experimental.pallas import tpu_sc as plsc

VOCAB = 1000000
D = 64
SCALE = math.sqrt(D)

NC = 2
NS = 16
NW = NC * NS

B = 4096 * 200
BPW = B // NW
C = 128
NCHUNK = BPW // C


def _body(idx_hbm, table_hbm, out_hbm, idx_v, rows_v, sem):
    wid = lax.axis_index("s") * NC + lax.axis_index("c")
    pltpu.sync_copy(idx_hbm.at[wid], idx_v)

    def chunk(j, carry):
        pltpu.async_copy(table_hbm.at[idx_v.at[j]], rows_v, sem).wait()

        def row(i, c2):
            for k4 in range(D // 16):
                sl = pl.ds(k4 * 16, 16)
                rows_v[i, sl] = rows_v[i, sl] * SCALE
            return c2

        lax.fori_loop(0, C, row, 0, unroll=4)

        pltpu.sync_copy(rows_v, out_hbm.at[wid, j])
        return carry

    lax.fori_loop(0, NCHUNK, chunk, 0)


@functools.partial(jax.jit)
def _lookup(x_flat, weight):
    mesh = plsc.VectorSubcoreMesh(core_axis_name="c", subcore_axis_name="s")
    f = pl.kernel(
        _body,
        mesh=mesh,
        out_type=jax.ShapeDtypeStruct((NW, NCHUNK, C, D), jnp.float32),
        scratch_types=[
            pltpu.VMEM((NCHUNK, C), jnp.int32),
            pltpu.VMEM((C, D), jnp.float32),
            pltpu.SemaphoreType.DMA,
        ],
        compiler_params=pltpu.CompilerParams(use_tc_tiling_on_sc=False),
    )
    return f(x_flat, weight)


def kernel(x, weight):
    xf = x.reshape(NW, NCHUNK, C).astype(jnp.int32)
    out = _lookup(xf, weight)
    return out.reshape(4096, 200, D)


# speedup vs baseline: 1.1639x; 1.1639x over previous
import functools
import math

import jax
import jax.numpy as jnp
from jax import lax
from jax.experimental import pallas as pl
from jax.experimental.pallas import tpu as pltpu
from jax.experimental.pallas import tpu_sc as plsc

VOCAB = 1000000
D = 64
SCALE = math.sqrt(D)

NC = 2
NS = 16
NW = NC * NS

B = 4096 * 200
BPW = B // NW
CH = 128
GPC = 2
C = CH * GPC
NCHUNK = BPW // C
NB = 2


def _body(idx_hbm, table_hbm, out_hbm, idx_v, rows_g, rows_s, g0, g1, s0, s1):
    wid = lax.axis_index("s") * NC + lax.axis_index("c")
    gsem = (g0, g1)
    ssem = (s0, s1)

    pltpu.sync_copy(idx_hbm.at[wid], idx_v)

    def issue_gather(j, b):
        for g in range(GPC):
            pltpu.async_copy(
                table_hbm.at[idx_v.at[j, g]],
                rows_g.at[b, pl.ds(g * CH, CH)],
                gsem[b],
            )

    def wait_gather(b):
        pltpu.make_async_copy(out_hbm.at[wid, 0], rows_g.at[b], gsem[b]).wait()

    def issue_scatter(j, b):
        pltpu.async_copy(rows_s.at[b], out_hbm.at[wid, j], ssem[b])

    def wait_scatter(b):
        pltpu.make_async_copy(rows_s.at[b], out_hbm.at[wid, 0], ssem[b]).wait()

    def scale(b):
        @plsc.parallel_loop(0, C, 1, unroll=8)
        def _(i):
            for k in range(D // 16):
                sl = pl.ds(k * 16, 16)
                rows_s[b, i, sl] = rows_g[b, i, sl] * SCALE

    for b in range(NB):
        issue_gather(b, b)
    for b in range(NB):
        wait_gather(b)
        scale(b)
        issue_scatter(b, b)
        issue_gather(b + NB, b)

    def outer(g, carry):
        for b in range(NB):
            j = NB + g * NB + b
            wait_gather(b)
            wait_scatter(b)
            scale(b)
            issue_scatter(j, b)
            issue_gather(j + NB, b)
        return carry

    lax.fori_loop(0, (NCHUNK - 2 * NB) // NB, outer, 0)

    for b in range(NB):
        j = NCHUNK - NB + b
        wait_gather(b)
        wait_scatter(b)
        scale(b)
        issue_scatter(j, b)
    for b in range(NB):
        wait_scatter(b)


@jax.jit
def _lookup(x_idx, weight):
    mesh = plsc.VectorSubcoreMesh(core_axis_name="c", subcore_axis_name="s")
    f = pl.kernel(
        _body,
        mesh=mesh,
        out_type=jax.ShapeDtypeStruct((NW, NCHUNK, C, D), jnp.float32),
        scratch_types=[
            pltpu.VMEM((NCHUNK, GPC, CH), jnp.int32),
            pltpu.VMEM((NB, C, D), jnp.float32),
            pltpu.VMEM((NB, C, D), jnp.float32),
            pltpu.SemaphoreType.DMA,
            pltpu.SemaphoreType.DMA,
            pltpu.SemaphoreType.DMA,
            pltpu.SemaphoreType.DMA,
        ],
        compiler_params=pltpu.CompilerParams(use_tc_tiling_on_sc=False),
    )
    return f(x_idx, weight)


def kernel(x, weight):
    xf = x.reshape(NW, NCHUNK, GPC, CH).astype(jnp.int32)
    out = _lookup(xf, weight)
    return out.reshape(4096, 200, D)
